# baseline (speedup 1.0000x reference)
import jax
import jax.numpy as jnp
from jax.experimental import pallas as pl
from jax.experimental.pallas import tpu as pltpu

T_B = 0.9
_EPS_ADJ = 1e-6
_EPS_DEN = 1e-6
_N = 900
_NP = 1024
_CHUNK = 128
_NCH = _NP // _CHUNK
_G = 1


def _body(bb8_ref, crows_ref, adj_ref, agg_ref, ab_s):
    jvecs = []
    for g in range(_G):
        crows = crows_ref[g]
        x1j = crows[0:1, :] - 0.5 * crows[2:3, :]
        y1j = crows[1:2, :] - 0.5 * crows[3:4, :]
        x2j = crows[0:1, :] + 0.5 * crows[2:3, :]
        y2j = crows[1:2, :] + 0.5 * crows[3:4, :]
        area_j = (x2j - x1j) * (y2j - y1j)
        jvecs.append((x1j, y1j, x2j, y2j, area_j))

    def giou_chunk(k, carry):
        r0 = k * _CHUNK
        sums = []
        for g in range(_G):
            x1j, y1j, x2j, y2j, area_j = jvecs[g]
            c = bb8_ref[g, pl.ds(r0, _CHUNK), :]
            x1i = c[:, 0:1] - 0.5 * c[:, 2:3]
            y1i = c[:, 1:2] - 0.5 * c[:, 3:4]
            x2i = c[:, 0:1] + 0.5 * c[:, 2:3]
            y2i = c[:, 1:2] + 0.5 * c[:, 3:4]
            area_i = (x2i - x1i) * (y2i - y1i)
            acc = jnp.zeros((_CHUNK, 128), jnp.float32)
            for w in range(_NCH):
                c0, c1 = w * 128, (w + 1) * 128
                x1s, y1s = x1j[:, c0:c1], y1j[:, c0:c1]
                x2s, y2s = x2j[:, c0:c1], y2j[:, c0:c1]
                area_s = area_j[:, c0:c1]
                wx = jnp.maximum(jnp.minimum(x2i, x2s) - jnp.maximum(x1i, x1s), 0.0)
                wy = jnp.maximum(jnp.minimum(y2i, y2s) - jnp.maximum(y1i, y1s), 0.0)
                inter = wx * wy
                union = area_i + area_s - inter
                iou = inter / union
                ew = jnp.maximum(jnp.maximum(x2i, x2s) - jnp.minimum(x1i, x1s), 0.0)
                eh = jnp.maximum(jnp.maximum(y2i, y2s) - jnp.minimum(y1i, y1s), 0.0)
                area_e = ew * eh
                giou = iou - (area_e - union) / area_e
                m = jnp.where(giou > T_B, 1.0, 0.0)
                ab_s[g, pl.ds(r0, _CHUNK), c0:c1] = m.astype(jnp.bfloat16)
                acc = acc + m
            sums.append(jnp.sum(acc))
        return tuple(carry[i] + sums[i] for i in range(_G))

    s0 = jax.lax.fori_loop(
        0, _NCH, giou_chunk, (jnp.float32(0.0),) * _G)

    def sweep_cond(st):
        t, _, changed = st
        any_changed = changed[0]
        for g in range(1, _G):
            any_changed = jnp.logical_or(any_changed, changed[g])
        return jnp.logical_and(any_changed, t < _N)

    def sweep(st):
        t, prev, _ = st

        def chunk(k, carry):
            r0 = k * _CHUNK
            sums = []
            for g in range(_G):
                lhs = ab_s[g, pl.ds(r0, _CHUNK), :]
                cnt = jax.lax.dot_general(
                    lhs, ab_s[g], (((1,), (0,)), ((), ())),
                    preferred_element_type=jnp.float32)
                new = jnp.where(
                    lhs.astype(jnp.float32) + cnt > _EPS_ADJ, 1.0, 0.0)
                ab_s[g, pl.ds(r0, _CHUNK), :] = new.astype(jnp.bfloat16)
                sums.append(jnp.sum(new))
            return tuple(carry[i] + sums[i] for i in range(_G))

        ns = jax.lax.fori_loop(0, _NCH, chunk, (jnp.float32(0.0),) * _G)
        return (t + 1, ns, tuple(ns[g] > prev[g] for g in range(_G)))

    jax.lax.while_loop(
        sweep_cond, sweep,
        (jnp.int32(0), s0, (jnp.bool_(True),) * _G))

    for g in range(_G):
        bb16 = bb8_ref[g].astype(jnp.bfloat16)
        m8 = jax.lax.dot_general(
            ab_s[g], bb16, (((1,), (0,)), ((), ())),
            preferred_element_type=jnp.float32)
        adj_ref[g] = ab_s[g, 0:_N, 0:_N].astype(jnp.float32)
        agg_ref[g] = m8[0:_N, 0:4] / (m8[0:_N, 4:5] + _EPS_DEN)


def kernel(bboxes, logits):
    B, n, _ = bboxes.shape
    f32 = jnp.float32
    bbp = jnp.pad(bboxes.astype(f32), ((0, 0), (0, _NP - n), (0, 0)))
    bb8 = jnp.concatenate(
        [bbp, jnp.ones((B, _NP, 1), f32), jnp.zeros((B, _NP, 3), f32)], axis=-1)
    crows = jnp.pad(jnp.swapaxes(bbp, 1, 2), ((0, 0), (0, 4), (0, 0)))

    adj, agg = pl.pallas_call(
        _body,
        grid=(B // _G,),
        in_specs=[
            pl.BlockSpec((_G, _NP, 8), lambda b: (b, 0, 0)),
            pl.BlockSpec((_G, 8, _NP), lambda b: (b, 0, 0)),
        ],
        out_specs=[
            pl.BlockSpec((_G, n, n), lambda b: (b, 0, 0)),
            pl.BlockSpec((_G, n, 4), lambda b: (b, 0, 0)),
        ],
        out_shape=[
            jax.ShapeDtypeStruct((B, n, n), f32),
            jax.ShapeDtypeStruct((B, n, 4), f32),
        ],
        scratch_shapes=[pltpu.VMEM((_G, _NP, _NP), jnp.bfloat16)],
        compiler_params=pltpu.CompilerParams(
            dimension_semantics=("parallel",),
            vmem_limit_bytes=100 * 1024 * 1024,
        ),
    )(bb8, crows)
    return (agg, logits, adj)


# speedup vs baseline: 1.0776x; 1.0776x over previous
import jax
import jax.numpy as jnp
from jax.experimental import pallas as pl
from jax.experimental.pallas import tpu as pltpu

T_B = 0.9
_EPS_ADJ = 1e-6
_EPS_DEN = 1e-6
_N = 900
_NP = 1024
_CHUNK = 128
_NCH = _NP // _CHUNK
_G = 2


def _body(bb8_ref, crows_ref, adj_ref, agg_ref, ab_s):
    jvecs = []
    for g in range(_G):
        crows = crows_ref[g]
        x1j = crows[0:1, :] - 0.5 * crows[2:3, :]
        y1j = crows[1:2, :] - 0.5 * crows[3:4, :]
        x2j = crows[0:1, :] + 0.5 * crows[2:3, :]
        y2j = crows[1:2, :] + 0.5 * crows[3:4, :]
        area_j = (x2j - x1j) * (y2j - y1j)
        jvecs.append((x1j, y1j, x2j, y2j, area_j))

    def giou_chunk(k, carry):
        r0 = k * _CHUNK
        sums = []
        for g in range(_G):
            x1j, y1j, x2j, y2j, area_j = jvecs[g]
            c = bb8_ref[g, pl.ds(r0, _CHUNK), :]
            x1i = c[:, 0:1] - 0.5 * c[:, 2:3]
            y1i = c[:, 1:2] - 0.5 * c[:, 3:4]
            x2i = c[:, 0:1] + 0.5 * c[:, 2:3]
            y2i = c[:, 1:2] + 0.5 * c[:, 3:4]
            area_i = (x2i - x1i) * (y2i - y1i)
            acc = jnp.zeros((_CHUNK, 128), jnp.float32)
            for w in range(_NCH):
                c0, c1 = w * 128, (w + 1) * 128
                x1s, y1s = x1j[:, c0:c1], y1j[:, c0:c1]
                x2s, y2s = x2j[:, c0:c1], y2j[:, c0:c1]
                area_s = area_j[:, c0:c1]
                wx = jnp.maximum(jnp.minimum(x2i, x2s) - jnp.maximum(x1i, x1s), 0.0)
                wy = jnp.maximum(jnp.minimum(y2i, y2s) - jnp.maximum(y1i, y1s), 0.0)
                inter = wx * wy
                union = area_i + area_s - inter
                iou = inter / union
                ew = jnp.maximum(jnp.maximum(x2i, x2s) - jnp.minimum(x1i, x1s), 0.0)
                eh = jnp.maximum(jnp.maximum(y2i, y2s) - jnp.minimum(y1i, y1s), 0.0)
                area_e = ew * eh
                giou = iou - (area_e - union) / area_e
                m = jnp.where(giou > T_B, 1.0, 0.0)
                ab_s[g, pl.ds(r0, _CHUNK), c0:c1] = m.astype(jnp.bfloat16)
                acc = acc + m
            sums.append(jnp.sum(acc))
        return tuple(carry[i] + sums[i] for i in range(_G))

    s0 = jax.lax.fori_loop(
        0, _NCH, giou_chunk, (jnp.float32(0.0),) * _G)

    def sweep_cond(st):
        t, _, changed = st
        any_changed = changed[0]
        for g in range(1, _G):
            any_changed = jnp.logical_or(any_changed, changed[g])
        return jnp.logical_and(any_changed, t < _N)

    def sweep(st):
        t, prev, _ = st

        def chunk(k, carry):
            r0 = k * _CHUNK
            sums = []
            for g in range(_G):
                lhs = ab_s[g, pl.ds(r0, _CHUNK), :]
                cnt = jax.lax.dot_general(
                    lhs, ab_s[g], (((1,), (0,)), ((), ())),
                    preferred_element_type=jnp.float32)
                new = jnp.where(
                    lhs.astype(jnp.float32) + cnt > _EPS_ADJ, 1.0, 0.0)
                ab_s[g, pl.ds(r0, _CHUNK), :] = new.astype(jnp.bfloat16)
                sums.append(jnp.sum(new))
            return tuple(carry[i] + sums[i] for i in range(_G))

        ns = jax.lax.fori_loop(0, _NCH, chunk, (jnp.float32(0.0),) * _G)
        return (t + 1, ns, tuple(ns[g] > prev[g] for g in range(_G)))

    jax.lax.while_loop(
        sweep_cond, sweep,
        (jnp.int32(0), s0, (jnp.bool_(True),) * _G))

    for g in range(_G):
        bb16 = bb8_ref[g].astype(jnp.bfloat16)
        m8 = jax.lax.dot_general(
            ab_s[g], bb16, (((1,), (0,)), ((), ())),
            preferred_element_type=jnp.float32)
        adj_ref[g] = ab_s[g, 0:_N, 0:_N].astype(jnp.float32)
        agg_ref[g] = m8[0:_N, 0:4] / (m8[0:_N, 4:5] + _EPS_DEN)


def kernel(bboxes, logits):
    B, n, _ = bboxes.shape
    f32 = jnp.float32
    bbp = jnp.pad(bboxes.astype(f32), ((0, 0), (0, _NP - n), (0, 0)))
    bb8 = jnp.concatenate(
        [bbp, jnp.ones((B, _NP, 1), f32), jnp.zeros((B, _NP, 3), f32)], axis=-1)
    crows = jnp.pad(jnp.swapaxes(bbp, 1, 2), ((0, 0), (0, 4), (0, 0)))

    adj, agg = pl.pallas_call(
        _body,
        grid=(B // _G,),
        in_specs=[
            pl.BlockSpec((_G, _NP, 8), lambda b: (b, 0, 0)),
            pl.BlockSpec((_G, 8, _NP), lambda b: (b, 0, 0)),
        ],
        out_specs=[
            pl.BlockSpec((_G, n, n), lambda b: (b, 0, 0)),
            pl.BlockSpec((_G, n, 4), lambda b: (b, 0, 0)),
        ],
        out_shape=[
            jax.ShapeDtypeStruct((B, n, n), f32),
            jax.ShapeDtypeStruct((B, n, 4), f32),
        ],
        scratch_shapes=[pltpu.VMEM((_G, _NP, _NP), jnp.bfloat16)],
        compiler_params=pltpu.CompilerParams(
            dimension_semantics=("parallel",),
            vmem_limit_bytes=100 * 1024 * 1024,
        ),
    )(bb8, crows)
    return (agg, logits, adj)


# speedup vs baseline: 1.1748x; 1.0902x over previous
import jax
import jax.numpy as jnp
from jax.experimental import pallas as pl
from jax.experimental.pallas import tpu as pltpu

T_B = 0.9
_EPS_ADJ = 1e-6
_EPS_DEN = 1e-6
_N = 900
_NP = 1024
_CHUNK = 128
_NCH = _NP // _CHUNK
_G = 2


def _body(bb8_ref, crows_ref, adj_ref, agg_ref, ab_s, cnt_s):
    jvecs = []
    for g in range(_G):
        crows = crows_ref[g]
        x1j = crows[0:1, :] - 0.5 * crows[2:3, :]
        y1j = crows[1:2, :] - 0.5 * crows[3:4, :]
        x2j = crows[0:1, :] + 0.5 * crows[2:3, :]
        y2j = crows[1:2, :] + 0.5 * crows[3:4, :]
        area_j = (x2j - x1j) * (y2j - y1j)
        jvecs.append((x1j, y1j, x2j, y2j, area_j))

    def giou_chunk(k, carry):
        r0 = k * _CHUNK
        sums = []
        for g in range(_G):
            x1j, y1j, x2j, y2j, area_j = jvecs[g]
            c = bb8_ref[g, pl.ds(r0, _CHUNK), :]
            x1i = c[:, 0:1] - 0.5 * c[:, 2:3]
            y1i = c[:, 1:2] - 0.5 * c[:, 3:4]
            x2i = c[:, 0:1] + 0.5 * c[:, 2:3]
            y2i = c[:, 1:2] + 0.5 * c[:, 3:4]
            area_i = (x2i - x1i) * (y2i - y1i)
            acc = jnp.zeros((_CHUNK, 128), jnp.float32)
            for w in range(_NCH):
                c0, c1 = w * 128, (w + 1) * 128
                x1s, y1s = x1j[:, c0:c1], y1j[:, c0:c1]
                x2s, y2s = x2j[:, c0:c1], y2j[:, c0:c1]
                area_s = area_j[:, c0:c1]
                wx = jnp.maximum(jnp.minimum(x2i, x2s) - jnp.maximum(x1i, x1s), 0.0)
                wy = jnp.maximum(jnp.minimum(y2i, y2s) - jnp.maximum(y1i, y1s), 0.0)
                inter = wx * wy
                union = area_i + area_s - inter
                iou = inter / union
                ew = jnp.maximum(jnp.maximum(x2i, x2s) - jnp.minimum(x1i, x1s), 0.0)
                eh = jnp.maximum(jnp.maximum(y2i, y2s) - jnp.minimum(y1i, y1s), 0.0)
                area_e = ew * eh
                giou = iou - (area_e - union) / area_e
                m = jnp.where(giou > T_B, 1.0, 0.0)
                ab_s[g, pl.ds(r0, _CHUNK), c0:c1] = m.astype(jnp.bfloat16)
                acc = acc + m
            sums.append(jnp.sum(acc))
        return tuple(carry[i] + sums[i] for i in range(_G))

    s0 = jax.lax.fori_loop(
        0, _NCH, giou_chunk, (jnp.float32(0.0),) * _G)

    def sweep_cond(st):
        t, _, changed = st
        any_changed = changed[0]
        for g in range(1, _G):
            any_changed = jnp.logical_or(any_changed, changed[g])
        return jnp.logical_and(any_changed, t < _N)

    def sweep(st):
        t, prev, _ = st
        for g in range(_G):
            cnt_s[g] = jax.lax.dot_general(
                ab_s[g], ab_s[g], (((1,), (0,)), ((), ())),
                preferred_element_type=jnp.float32)

        def chunk(k, carry):
            r0 = k * _CHUNK
            sums = []
            for g in range(_G):
                old = ab_s[g, pl.ds(r0, _CHUNK), :]
                new = jnp.where(
                    old.astype(jnp.float32) + cnt_s[g, pl.ds(r0, _CHUNK), :]
                    > _EPS_ADJ, 1.0, 0.0)
                ab_s[g, pl.ds(r0, _CHUNK), :] = new.astype(jnp.bfloat16)
                sums.append(jnp.sum(new))
            return tuple(carry[i] + sums[i] for i in range(_G))

        ns = jax.lax.fori_loop(0, _NCH, chunk, (jnp.float32(0.0),) * _G)
        return (t + 1, ns, tuple(ns[g] > prev[g] for g in range(_G)))

    jax.lax.while_loop(
        sweep_cond, sweep,
        (jnp.int32(0), s0, (jnp.bool_(True),) * _G))

    for g in range(_G):
        bb16 = bb8_ref[g].astype(jnp.bfloat16)
        m8 = jax.lax.dot_general(
            ab_s[g], bb16, (((1,), (0,)), ((), ())),
            preferred_element_type=jnp.float32)
        adj_ref[g] = ab_s[g, 0:_N, 0:_N].astype(jnp.float32)
        agg_ref[g] = m8[0:_N, 0:4] / (m8[0:_N, 4:5] + _EPS_DEN)


def kernel(bboxes, logits):
    B, n, _ = bboxes.shape
    f32 = jnp.float32
    bbp = jnp.pad(bboxes.astype(f32), ((0, 0), (0, _NP - n), (0, 0)))
    bb8 = jnp.concatenate(
        [bbp, jnp.ones((B, _NP, 1), f32), jnp.zeros((B, _NP, 3), f32)], axis=-1)
    crows = jnp.pad(jnp.swapaxes(bbp, 1, 2), ((0, 0), (0, 4), (0, 0)))

    adj, agg = pl.pallas_call(
        _body,
        grid=(B // _G,),
        in_specs=[
            pl.BlockSpec((_G, _NP, 8), lambda b: (b, 0, 0)),
            pl.BlockSpec((_G, 8, _NP), lambda b: (b, 0, 0)),
        ],
        out_specs=[
            pl.BlockSpec((_G, n, n), lambda b: (b, 0, 0)),
            pl.BlockSpec((_G, n, 4), lambda b: (b, 0, 0)),
        ],
        out_shape=[
            jax.ShapeDtypeStruct((B, n, n), f32),
            jax.ShapeDtypeStruct((B, n, 4), f32),
        ],
        scratch_shapes=[
            pltpu.VMEM((_G, _NP, _NP), jnp.bfloat16),
            pltpu.VMEM((_G, _NP, _NP), jnp.float32),
        ],
        compiler_params=pltpu.CompilerParams(
            dimension_semantics=("parallel",),
            vmem_limit_bytes=100 * 1024 * 1024,
        ),
    )(bb8, crows)
    return (agg, logits, adj)
